# Initial kernel scaffold; baseline (speedup 1.0000x reference)
import jax
import jax.numpy as jnp
from jax.experimental import pallas as pl


def kernel(obs_features, feature_mask, feat_features, obs_adjs, data_x, poW1, pob1, pog, pobe, poW2, pob2, pfW1, pfb1, pfg, pfbe, pfW2, pfb2, gWsrc, gWdst, gWedge, gasrc, gadst, gaedge, gbias, opW1, opb1, opg, opbe, opW2, opb2):
    raise NotImplementedError("write your pallas kernel here")


# speedup vs baseline: 1805.6010x; 1805.6010x over previous
import functools

import jax
import jax.numpy as jnp
from jax.experimental import pallas as pl

_HEADS = 4
_CH = 32
_HID = 128
_ROW_BLOCK = 2000
_NEG = -1e30


def _dot(a, b):
    return jnp.dot(a, b, preferred_element_type=jnp.float32)


def _feat_kernel(ff_ref, pfW1_ref, pfb1_ref, pfg_ref, pfbe_ref, pfW2_ref,
                 pfb2_ref, gWsrc_ref, asrc_ref, onehotT_ref, gWdst_ref,
                 adst_selT_ref, eprod_ref,
                 xs_ref, als_ref, wdal_ref, cvec_ref):
    ff = ff_ref[...]
    h = _dot(ff, pfW1_ref[...]) + pfb1_ref[...]
    mu = jnp.mean(h, axis=0, keepdims=True)
    var = jnp.mean((h - mu) * (h - mu), axis=0, keepdims=True)
    hn = (h - mu) * jax.lax.rsqrt(var + 1e-5) * pfg_ref[...] + pfbe_ref[...]
    feat_h = jax.nn.relu(_dot(hn, pfW2_ref[...]) + pfb2_ref[...])
    xs = _dot(feat_h, gWsrc_ref[...])
    xs_ref[...] = xs
    als_ref[...] = _dot(xs * asrc_ref[...], onehotT_ref[...])
    wdal_ref[...] = _dot(gWdst_ref[...], adst_selT_ref[...])
    cvec_ref[...] = _dot(eprod_ref[...], onehotT_ref[...])


def _pass1_kernel(x_ref, W1_ref, b1_ref, h1_ref, stats_ref):
    i = pl.program_id(0)
    h = _dot(x_ref[...], W1_ref[...]) + b1_ref[...]
    h1_ref[...] = h

    @pl.when(i == 0)
    def _():
        stats_ref[...] = jnp.zeros_like(stats_ref)

    stats_ref[0:1, :] += jnp.sum(h, axis=0, keepdims=True)
    stats_ref[1:2, :] += jnp.sum(h * h, axis=0, keepdims=True)


def _pass2_kernel(h1_ref, dx_ref, mask_ref, stats1_ref, poW2_ref, pob2_ref,
                  pog_ref, pobe_ref, wdal_ref, alsT_ref, cvec_ref, xs_ref,
                  gbias_ref, opW1t_ref, opW1b_ref, opb1_ref,
                  h2_ref, stats2_ref, *, n_rows):
    i = pl.program_id(0)
    inv_n = 1.0 / n_rows
    mu = stats1_ref[0:1, :] * inv_n
    var = stats1_ref[1:2, :] * inv_n - mu * mu
    hn = (h1_ref[...] - mu) * jax.lax.rsqrt(var + 1e-5) * pog_ref[...] \
        + pobe_ref[...]
    obs_h = jax.nn.relu(_dot(hn, poW2_ref[...]) + pob2_ref[...])

    al_d = _dot(obs_h, wdal_ref[...])
    dx = dx_ref[...]
    mvalid = mask_ref[...]

    g_parts = []
    for h in range(_HEADS):
        c_h = cvec_ref[0, h]
        alpha = al_d[:, h:h + 1] + alsT_ref[h:h + 1, :] + dx * c_h
        alpha = jnp.where(alpha >= 0, alpha, 0.2 * alpha)
        alpha = jnp.where(mvalid > 0, alpha, _NEG)
        amax = jnp.max(alpha, axis=1, keepdims=True)
        amax = jnp.where(amax > 0.5 * _NEG, amax, 0.0)
        ex = jnp.exp(alpha - amax) * mvalid
        den = jnp.sum(ex, axis=1, keepdims=True)
        a = ex / (den + 1e-16)
        g_parts.append(_dot(a, xs_ref[:, h * _CH:(h + 1) * _CH]))
    g = jnp.concatenate(g_parts, axis=1) + gbias_ref[...]

    h2 = _dot(obs_h, opW1t_ref[...]) + _dot(g, opW1b_ref[...]) + opb1_ref[...]
    h2_ref[...] = h2

    @pl.when(i == 0)
    def _():
        stats2_ref[...] = jnp.zeros_like(stats2_ref)

    stats2_ref[0:1, :] += jnp.sum(h2, axis=0, keepdims=True)
    stats2_ref[1:2, :] += jnp.sum(h2 * h2, axis=0, keepdims=True)


def _pass3_kernel(h2_ref, stats2_ref, opg_ref, opbe_ref, opW2_ref, opb2_ref,
                  out_ref, *, n_rows):
    inv_n = 1.0 / n_rows
    mu = stats2_ref[0:1, :] * inv_n
    var = stats2_ref[1:2, :] * inv_n - mu * mu
    hn = (h2_ref[...] - mu) * jax.lax.rsqrt(var + 1e-5) * opg_ref[...] \
        + opbe_ref[...]
    out_ref[...] = jax.nn.relu(_dot(hn, opW2_ref[...]) + opb2_ref[...])


def _full(arr_shape):
    nd = len(arr_shape)
    return pl.BlockSpec(arr_shape, lambda i: (0,) * nd)


def kernel(obs_features, feature_mask, feat_features, obs_adjs, data_x,
           poW1, pob1, pog, pobe, poW2, pob2,
           pfW1, pfb1, pfg, pfbe, pfW2, pfb2,
           gWsrc, gWdst, gWedge, gasrc, gadst, gaedge, gbias,
           opW1, opb1, opg, opbe, opW2, opb2):
    n_obs, n_feat = feature_mask.shape
    f32 = jnp.float32

    row = lambda v: v.reshape(1, -1).astype(f32)
    pob1r, pob2r, pogr, pober = row(pob1), row(pob2), row(pog), row(pobe)
    pfb1r, pfb2r, pfgr, pfber = row(pfb1), row(pfb2), row(pfg), row(pfbe)
    opb1r, opb2r, opgr, opber = row(opb1), row(opb2), row(opg), row(opbe)
    gbiasr = row(gbias)

    onehot = (jnp.arange(_HID, dtype=jnp.int32)[None, :] // _CH ==
              jnp.arange(_HEADS, dtype=jnp.int32)[:, None]).astype(f32)
    onehotT = onehot.T
    asrc_flat = gasrc.reshape(1, _HID).astype(f32)
    adst_flat = gadst.reshape(1, _HID).astype(f32)
    adst_selT = (onehot * adst_flat).T
    eprod = (gWedge.reshape(1, _HID) * gaedge.reshape(1, _HID)).astype(f32)

    xs, al_s, wd_al, cvec = pl.pallas_call(
        _feat_kernel,
        out_shape=[
            jax.ShapeDtypeStruct((n_feat, _HID), f32),
            jax.ShapeDtypeStruct((n_feat, _HEADS), f32),
            jax.ShapeDtypeStruct((_HID, _HEADS), f32),
            jax.ShapeDtypeStruct((1, _HEADS), f32),
        ],
    )(feat_features, pfW1, pfb1r, pfgr, pfber, pfW2, pfb2r,
      gWsrc, asrc_flat, onehotT, gWdst, adst_selT, eprod)
    alsT = al_s.T

    nb = n_obs // _ROW_BLOCK
    R = _ROW_BLOCK
    d_obs = obs_features.shape[1]

    h1, stats1 = pl.pallas_call(
        _pass1_kernel,
        grid=(nb,),
        in_specs=[
            pl.BlockSpec((R, d_obs), lambda i: (i, 0)),
            _full(poW1.shape),
            _full(pob1r.shape),
        ],
        out_specs=[
            pl.BlockSpec((R, _HID), lambda i: (i, 0)),
            pl.BlockSpec((8, _HID), lambda i: (0, 0)),
        ],
        out_shape=[
            jax.ShapeDtypeStruct((n_obs, _HID), f32),
            jax.ShapeDtypeStruct((8, _HID), f32),
        ],
    )(obs_features, poW1, pob1r)

    maskf = feature_mask.astype(f32)
    opW1t = opW1[:_HID]
    opW1b = opW1[_HID:]

    h2, stats2 = pl.pallas_call(
        functools.partial(_pass2_kernel, n_rows=float(n_obs)),
        grid=(nb,),
        in_specs=[
            pl.BlockSpec((R, _HID), lambda i: (i, 0)),
            pl.BlockSpec((R, n_feat), lambda i: (i, 0)),
            pl.BlockSpec((R, n_feat), lambda i: (i, 0)),
            _full((8, _HID)),
            _full(poW2.shape),
            _full(pob2r.shape),
            _full(pogr.shape),
            _full(pober.shape),
            _full(wd_al.shape),
            _full(alsT.shape),
            _full(cvec.shape),
            _full(xs.shape),
            _full(gbiasr.shape),
            _full(opW1t.shape),
            _full(opW1b.shape),
            _full(opb1r.shape),
        ],
        out_specs=[
            pl.BlockSpec((R, _HID), lambda i: (i, 0)),
            pl.BlockSpec((8, _HID), lambda i: (0, 0)),
        ],
        out_shape=[
            jax.ShapeDtypeStruct((n_obs, _HID), f32),
            jax.ShapeDtypeStruct((8, _HID), f32),
        ],
    )(h1, data_x, maskf, stats1, poW2, pob2r, pogr, pober,
      wd_al, alsT, cvec, xs, gbiasr, opW1t, opW1b, opb1r)

    out = pl.pallas_call(
        functools.partial(_pass3_kernel, n_rows=float(n_obs)),
        grid=(nb,),
        in_specs=[
            pl.BlockSpec((R, _HID), lambda i: (i, 0)),
            _full((8, _HID)),
            _full(opgr.shape),
            _full(opber.shape),
            _full(opW2.shape),
            _full(opb2r.shape),
        ],
        out_specs=pl.BlockSpec((R, _HID), lambda i: (i, 0)),
        out_shape=jax.ShapeDtypeStruct((n_obs, _HID), f32),
    )(h2, stats2, opgr, opber, opW2, opb2r)

    return out


# speedup vs baseline: 2034.1311x; 1.1266x over previous
import jax
import jax.numpy as jnp
from jax.experimental import pallas as pl

_HEADS = 4
_CH = 32
_HID = 128
_NEG = -1e30


def _dot(a, b):
    return jnp.dot(a, b, preferred_element_type=jnp.float32)


def _bn(h, g, be):
    mu = jnp.mean(h, axis=0, keepdims=True)
    var = jnp.mean(h * h, axis=0, keepdims=True) - mu * mu
    return (h - mu) * jax.lax.rsqrt(var + 1e-5) * g + be


def _fused_kernel(obs_ref, dx_ref, mask_ref, ff_ref,
                  poW1_ref, pob1_ref, pog_ref, pobe_ref, poW2_ref, pob2_ref,
                  pfW1_ref, pfb1_ref, pfg_ref, pfbe_ref, pfW2_ref, pfb2_ref,
                  gWsrc_ref, asrc_sel_ref, onehotT_ref, wdal_ref, eprod_ref,
                  gbias_ref, opW1t_ref, opW1b_ref, opb1_ref, opg_ref,
                  opbe_ref, opW2_ref, opb2_ref,
                  out_ref):
    hf = _dot(ff_ref[...], pfW1_ref[...]) + pfb1_ref[...]
    feat_h = jax.nn.relu(
        _dot(_bn(hf, pfg_ref[...], pfbe_ref[...]), pfW2_ref[...])
        + pfb2_ref[...])
    xs = _dot(feat_h, gWsrc_ref[...])
    al_sT = jax.lax.dot_general(
        asrc_sel_ref[...], xs, (((1,), (1,)), ((), ())),
        preferred_element_type=jnp.float32)
    cvec = _dot(eprod_ref[...], onehotT_ref[...])

    h1 = _dot(obs_ref[...], poW1_ref[...]) + pob1_ref[...]
    obs_h = jax.nn.relu(
        _dot(_bn(h1, pog_ref[...], pobe_ref[...]), poW2_ref[...])
        + pob2_ref[...])

    al_d = _dot(obs_h, wdal_ref[...])
    dx = dx_ref[...]
    mvalid = mask_ref[...].astype(jnp.float32)

    g_parts = []
    for h in range(_HEADS):
        c_h = cvec[0, h]
        alpha = al_d[:, h:h + 1] + al_sT[h:h + 1, :] + dx * c_h
        alpha = jnp.where(alpha >= 0, alpha, 0.2 * alpha)
        alpha = jnp.where(mvalid > 0, alpha, _NEG)
        amax = jnp.max(alpha, axis=1, keepdims=True)
        amax = jnp.where(amax > 0.5 * _NEG, amax, 0.0)
        ex = jnp.exp(alpha - amax) * mvalid
        den = jnp.sum(ex, axis=1, keepdims=True)
        a = ex / (den + 1e-16)
        g_parts.append(_dot(a, xs[:, h * _CH:(h + 1) * _CH]))
    g = jnp.concatenate(g_parts, axis=1) + gbias_ref[...]

    h2 = _dot(obs_h, opW1t_ref[...]) + _dot(g, opW1b_ref[...]) + opb1_ref[...]
    out_ref[...] = jax.nn.relu(
        _dot(_bn(h2, opg_ref[...], opbe_ref[...]), opW2_ref[...])
        + opb2_ref[...])


def kernel(obs_features, feature_mask, feat_features, obs_adjs, data_x,
           poW1, pob1, pog, pobe, poW2, pob2,
           pfW1, pfb1, pfg, pfbe, pfW2, pfb2,
           gWsrc, gWdst, gWedge, gasrc, gadst, gaedge, gbias,
           opW1, opb1, opg, opbe, opW2, opb2):
    n_obs, n_feat = feature_mask.shape
    f32 = jnp.float32

    row = lambda v: v.reshape(1, -1).astype(f32)

    onehot = (jnp.arange(_HID, dtype=jnp.int32)[None, :] // _CH ==
              jnp.arange(_HEADS, dtype=jnp.int32)[:, None]).astype(f32)
    onehotT = onehot.T
    asrc_sel = onehot * gasrc.reshape(1, _HID).astype(f32)
    adst_flat = gadst.reshape(1, _HID).astype(f32)
    wd_al = gWdst.astype(f32) @ (onehot * adst_flat).T
    eprod = (gWedge.reshape(1, _HID) * gaedge.reshape(1, _HID)).astype(f32)

    out = pl.pallas_call(
        _fused_kernel,
        out_shape=jax.ShapeDtypeStruct((n_obs, _HID), f32),
    )(obs_features, data_x, feature_mask, feat_features,
      poW1, row(pob1), row(pog), row(pobe), poW2, row(pob2),
      pfW1, row(pfb1), row(pfg), row(pfbe), pfW2, row(pfb2),
      gWsrc, asrc_sel, onehotT, wd_al, eprod,
      row(gbias), opW1[:_HID], opW1[_HID:], row(opb1), row(opg),
      row(opbe), opW2, row(opb2))
    return out


# speedup vs baseline: 2534.0039x; 1.2457x over previous
import jax
import jax.numpy as jnp
from jax.experimental import pallas as pl

_HEADS = 4
_CH = 32
_HID = 128
_NEG = -1e30


def _dot(a, b):
    return jnp.dot(a, b, preferred_element_type=jnp.float32)


def _bn_scale_shift(h, g, be):
    mu = jnp.mean(h, axis=0, keepdims=True)
    var = jnp.mean(h * h, axis=0, keepdims=True) - mu * mu
    s = jax.lax.rsqrt(var + 1e-5) * g
    return s, be - mu * s


def _fused_kernel(obs_ref, dx_ref, mask_ref, ff_ref,
                  poW1_ref, pob1_ref, pog_ref, pobe_ref, poW2_ref, pob2_ref,
                  pfW1_ref, pfb1_ref, pfg_ref, pfbe_ref, pfW2_ref, pfb2_ref,
                  gWsrc_ref, asrc_sel_ref, onehotT_ref, wdal_ref, eprod_ref,
                  gbias_ref, opW1t_ref, opW1b_ref, opb1_ref, opg_ref,
                  opbe_ref, opW2_ref, opb2_ref,
                  out_ref):
    hf = _dot(ff_ref[...], pfW1_ref[...]) + pfb1_ref[...]
    sf, tf = _bn_scale_shift(hf, pfg_ref[...], pfbe_ref[...])
    feat_h = jax.nn.relu(_dot(hf * sf + tf, pfW2_ref[...]) + pfb2_ref[...])
    xs = _dot(feat_h, gWsrc_ref[...])
    al_sT = jax.lax.dot_general(
        asrc_sel_ref[...], xs, (((1,), (1,)), ((), ())),
        preferred_element_type=jnp.float32)
    cvec = _dot(eprod_ref[...], onehotT_ref[...])

    h1 = _dot(obs_ref[...], poW1_ref[...]) + pob1_ref[...]
    s1, t1 = _bn_scale_shift(h1, pog_ref[...], pobe_ref[...])
    obs_h = jax.nn.relu(_dot(h1 * s1 + t1, poW2_ref[...]) + pob2_ref[...])

    ones_col = jnp.ones((xs.shape[0], 1), dtype=jnp.float32)
    xs_augs = [
        jnp.concatenate([xs[:, h * _CH:(h + 1) * _CH], ones_col], axis=1)
        for h in range(_HEADS)]
    n_rows = obs_ref.shape[0]
    n_chunks = 5
    rc = n_rows // n_chunks

    h2_parts = []
    for r in range(n_chunks):
        sl = pl.ds(r * rc, rc)
        obs_c = jax.lax.slice(obs_h, (r * rc, 0), ((r + 1) * rc, _HID))
        al_d = _dot(obs_c, wdal_ref[...])
        dx = dx_ref[sl, :]
        mask = mask_ref[sl, :] != 0
        g_parts = []
        for h in range(_HEADS):
            c_h = cvec[0, h]
            raw = al_d[:, h:h + 1] + al_sT[h:h + 1, :] + dx * c_h
            raw = jnp.maximum(raw, 0.2 * raw)
            alpha = jnp.where(mask, raw, _NEG)
            amax = jnp.max(alpha, axis=1, keepdims=True)
            amax = jnp.where(amax > 0.5 * _NEG, amax, 0.0)
            ex = jnp.exp(alpha - amax)
            res = _dot(ex, xs_augs[h])
            rec = 1.0 / (res[:, _CH:_CH + 1] + 1e-16)
            g_parts.append(res[:, :_CH] * rec)
        g = jnp.concatenate(g_parts, axis=1) + gbias_ref[...]
        h2_parts.append(_dot(obs_c, opW1t_ref[...]) + _dot(g, opW1b_ref[...])
                        + opb1_ref[...])

    h2 = jnp.concatenate(h2_parts, axis=0)
    s2, t2 = _bn_scale_shift(h2, opg_ref[...], opbe_ref[...])
    out_ref[...] = jax.nn.relu(_dot(h2 * s2 + t2, opW2_ref[...])
                               + opb2_ref[...])


def kernel(obs_features, feature_mask, feat_features, obs_adjs, data_x,
           poW1, pob1, pog, pobe, poW2, pob2,
           pfW1, pfb1, pfg, pfbe, pfW2, pfb2,
           gWsrc, gWdst, gWedge, gasrc, gadst, gaedge, gbias,
           opW1, opb1, opg, opbe, opW2, opb2):
    n_obs, n_feat = feature_mask.shape
    f32 = jnp.float32

    row = lambda v: v.reshape(1, -1).astype(f32)

    onehot = (jnp.arange(_HID, dtype=jnp.int32)[None, :] // _CH ==
              jnp.arange(_HEADS, dtype=jnp.int32)[:, None]).astype(f32)
    onehotT = onehot.T
    asrc_sel = onehot * gasrc.reshape(1, _HID).astype(f32)
    adst_flat = gadst.reshape(1, _HID).astype(f32)
    wd_al = gWdst.astype(f32) @ (onehot * adst_flat).T
    eprod = (gWedge.reshape(1, _HID) * gaedge.reshape(1, _HID)).astype(f32)

    out = pl.pallas_call(
        _fused_kernel,
        out_shape=jax.ShapeDtypeStruct((n_obs, _HID), f32),
        input_output_aliases={0: 0},
    )(obs_features, data_x, feature_mask.astype(jnp.int8), feat_features,
      poW1, row(pob1), row(pog), row(pobe), poW2, row(pob2),
      pfW1, row(pfb1), row(pfg), row(pfbe), pfW2, row(pfb2),
      gWsrc, asrc_sel, onehotT, wd_al, eprod,
      row(gbias), opW1[:_HID], opW1[_HID:], row(opb1), row(opg),
      row(opbe), opW2, row(opb2))
    return out


# speedup vs baseline: 4320.4254x; 1.7050x over previous
import jax
import jax.numpy as jnp
from jax.experimental import pallas as pl

_HEADS = 4
_CH = 32
_HID = 128
_NEG = -1e30


def _dot(a, b):
    return jnp.dot(a, b, preferred_element_type=jnp.float32)


def _dot_t(a, b, dims):
    return jax.lax.dot_general(a, b, (dims, ((), ())),
                               preferred_element_type=jnp.float32)


def _bn_scale_shift(h, g, be):
    mu = jnp.mean(h, axis=0, keepdims=True)
    var = jnp.mean(h * h, axis=0, keepdims=True) - mu * mu
    s = jax.lax.rsqrt(var + 1e-5) * g
    return s, be - mu * s


def _fused_kernel(obs_ref, dxT_ref, maskT_ref, ff_ref,
                  poW1_ref, pob1_ref, pog_ref, pobe_ref, poW2_ref, pob2_ref,
                  pfW1_ref, pfb1_ref, pfg_ref, pfbe_ref, pfW2_ref, pfb2_ref,
                  gWsrc_ref, asrc_row_ref, onehotT_ref, wdalT_ref, eprod_ref,
                  gbias_ref, opW1t_ref, opW1b_ref, opb1_ref, opg_ref,
                  opbe_ref, opW2_ref, opb2_ref,
                  out_ref):
    hf = _dot(ff_ref[...], pfW1_ref[...]) + pfb1_ref[...]
    sf, tf = _bn_scale_shift(hf, pfg_ref[...], pfbe_ref[...])
    feat_h = jax.nn.relu(_dot(hf * sf + tf, pfW2_ref[...]) + pfb2_ref[...])
    xs = _dot(feat_h, gWsrc_ref[...])
    al_s = _dot(xs * asrc_row_ref[...], onehotT_ref[...])
    cvec = _dot(eprod_ref[...], onehotT_ref[...])
    ones_col = jnp.ones((xs.shape[0], 1), dtype=jnp.float32)
    xs_augs = [
        jnp.concatenate([xs[:, h * _CH:(h + 1) * _CH], ones_col], axis=1)
        for h in range(_HEADS)]

    h1 = _dot(obs_ref[...], poW1_ref[...]) + pob1_ref[...]
    s1, t1 = _bn_scale_shift(h1, pog_ref[...], pobe_ref[...])
    obs_h = jax.nn.relu(_dot(h1 * s1 + t1, poW2_ref[...]) + pob2_ref[...])

    bias2 = _dot(gbias_ref[...], opW1b_ref[...]) + opb1_ref[...]

    n_rows = obs_ref.shape[0]
    n_chunks = 5
    rc = n_rows // n_chunks

    h2_parts = []
    for r in range(n_chunks):
        obs_c = jax.lax.slice(obs_h, (r * rc, 0), ((r + 1) * rc, _HID))
        al_dT = _dot_t(wdalT_ref[...], obs_c, ((1,), (1,)))
        dxT = dxT_ref[:, pl.ds(r * rc, rc)]
        maskT = maskT_ref[:, pl.ds(r * rc, rc)] != 0
        gT_parts = []
        for h in range(_HEADS):
            c_h = cvec[0, h]
            raw = al_dT[h:h + 1, :] + al_s[:, h:h + 1] + dxT * c_h
            raw = jnp.maximum(raw, 0.2 * raw)
            alpha = jnp.where(maskT, raw, _NEG)
            amax = jnp.max(alpha, axis=0, keepdims=True)
            amax = jnp.where(amax > 0.5 * _NEG, amax, 0.0)
            ex = jnp.exp(alpha - amax)
            res = _dot_t(xs_augs[h], ex, ((0,), (0,)))
            rec = 1.0 / (res[_CH:_CH + 1, :] + 1e-16)
            gT_parts.append(res[:_CH, :] * rec)
        gT = jnp.concatenate(gT_parts, axis=0)
        h2_parts.append(_dot(obs_c, opW1t_ref[...])
                        + _dot_t(gT, opW1b_ref[...], ((0,), (0,)))
                        + bias2)

    h2 = jnp.concatenate(h2_parts, axis=0)
    s2, t2 = _bn_scale_shift(h2, opg_ref[...], opbe_ref[...])
    out_ref[...] = jax.nn.relu(_dot(h2 * s2 + t2, opW2_ref[...])
                               + opb2_ref[...])


def kernel(obs_features, feature_mask, feat_features, obs_adjs, data_x,
           poW1, pob1, pog, pobe, poW2, pob2,
           pfW1, pfb1, pfg, pfbe, pfW2, pfb2,
           gWsrc, gWdst, gWedge, gasrc, gadst, gaedge, gbias,
           opW1, opb1, opg, opbe, opW2, opb2):
    n_obs, n_feat = feature_mask.shape
    f32 = jnp.float32

    row = lambda v: v.reshape(1, -1).astype(f32)

    onehot = (jnp.arange(_HID, dtype=jnp.int32)[None, :] // _CH ==
              jnp.arange(_HEADS, dtype=jnp.int32)[:, None]).astype(f32)
    onehotT = onehot.T
    asrc_row = gasrc.reshape(1, _HID).astype(f32)
    adst_flat = gadst.reshape(1, _HID).astype(f32)
    wd_alT = (onehot * adst_flat) @ gWdst.astype(f32).T
    eprod = (gWedge.reshape(1, _HID) * gaedge.reshape(1, _HID)).astype(f32)

    out = pl.pallas_call(
        _fused_kernel,
        out_shape=jax.ShapeDtypeStruct((n_obs, _HID), f32),
    )(obs_features, data_x.T, feature_mask.astype(jnp.int8).T, feat_features,
      poW1, row(pob1), row(pog), row(pobe), poW2, row(pob2),
      pfW1, row(pfb1), row(pfg), row(pfbe), pfW2, row(pfb2),
      gWsrc, asrc_row, onehotT, wd_alT, eprod,
      row(gbias), opW1[:_HID], opW1[_HID:], row(opb1), row(opg),
      row(opbe), opW2, row(opb2))
    return out
